# baseline (speedup 1.0000x reference)
import jax
import jax.numpy as jnp
from jax.experimental import pallas as pl
from jax.experimental.pallas import tpu as pltpu

B = 16
C = 256
BC = B * C
LEVEL_HW = (4096, 1024, 256, 64)
S_TOTAL = 5440
S_TILE = 256
N_TILES = 22
TILE_LEVEL = (0,) * 16 + (1,) * 4 + (2, 3)


def _kern(x, embt, out_ref):
    out_ref[...] = x[...].T + embt[0]


def kernel(feats_0, feats_1, feats_2, feats_3, level_start_idx, spatial_shapes, embed_weight):
    xcat = jnp.concatenate(
        [
            f.reshape(BC, hw)
            for f, hw in zip((feats_0, feats_1, feats_2, feats_3), LEVEL_HW)
        ],
        axis=1,
    )
    emb_bc = jnp.tile(embed_weight, (1, B))
    emb_tiles = emb_bc[jnp.asarray(TILE_LEVEL, dtype=jnp.int32)].reshape(N_TILES, 1, BC)
    out = pl.pallas_call(
        _kern,
        grid=(N_TILES,),
        in_specs=[
            pl.BlockSpec((BC, S_TILE), lambda i: (0, i)),
            pl.BlockSpec((1, 1, BC), lambda i: (i, 0, 0)),
        ],
        out_specs=pl.BlockSpec((S_TILE, BC), lambda i: (i, 0)),
        out_shape=jax.ShapeDtypeStruct((S_TOTAL, BC), jnp.float32),
        compiler_params=pltpu.CompilerParams(
            dimension_semantics=("parallel",),
        ),
    )(xcat, emb_tiles)
    return out.reshape(S_TOTAL, B, C)


# speedup vs baseline: 1.1885x; 1.1885x over previous
import jax
import jax.numpy as jnp
from jax.experimental import pallas as pl
from jax.experimental.pallas import tpu as pltpu

B = 16
C = 256
BC = B * C
LEVEL_HW = (4096, 1024, 256, 64)
S_TOTAL = 5440
S_TILE = 512


def _kern(f0, f1, f2, f3, emb, out_ref):
    i = pl.program_id(0)

    @pl.when(i < 8)
    def _():
        out_ref[...] = f0[...].T + emb[0][None, :]

    @pl.when((i >= 8) & (i < 10))
    def _():
        out_ref[...] = f1[...].T + emb[1][None, :]

    @pl.when(i == 10)
    def _():
        out_ref[0:256, :] = f2[...].T + emb[2][None, :]
        out_ref[256:320, :] = f3[...].T + emb[3][None, :]


def kernel(feats_0, feats_1, feats_2, feats_3, level_start_idx, spatial_shapes, embed_weight):
    feats = [
        f.reshape(BC, hw)
        for f, hw in zip((feats_0, feats_1, feats_2, feats_3), LEVEL_HW)
    ]
    emb_bc = jnp.tile(embed_weight, (1, B))
    in_specs = [
        pl.BlockSpec((BC, 512), lambda i: (0, jnp.clip(i, 0, 7))),
        pl.BlockSpec((BC, 512), lambda i: (0, jnp.clip(i - 8, 0, 1))),
        pl.BlockSpec((BC, 256), lambda i: (0, 0)),
        pl.BlockSpec((BC, 64), lambda i: (0, 0)),
        pl.BlockSpec((4, BC), lambda i: (0, 0)),
    ]
    out = pl.pallas_call(
        _kern,
        grid=(11,),
        in_specs=in_specs,
        out_specs=pl.BlockSpec((S_TILE, BC), lambda i: (i, 0)),
        out_shape=jax.ShapeDtypeStruct((S_TOTAL, BC), jnp.float32),
        compiler_params=pltpu.CompilerParams(
            dimension_semantics=("parallel",),
            vmem_limit_bytes=110 * 1024 * 1024,
        ),
    )(*feats, emb_bc)
    return out.reshape(S_TOTAL, B, C)
